# baseline (speedup 1.0000x reference)
import functools

import jax
import jax.numpy as jnp
from jax import lax
from jax.experimental import pallas as pl
from jax.experimental.pallas import tpu as pltpu
from jax.experimental.pallas import tpu_sc as plsc

V = 50304
T = 2048
C = 768
NH = 12
HD = C // NH
DFF = 4 * C
E = 8
TOPK = 2

EG = 8
RB = 1024
VB = 384

BLKM = 128
NBLK = (TOPK * T) // BLKM + E
PADN = NBLK * BLKM
MG = 16
CG = 8


def _ln(x, g):
    m = jnp.mean(x, axis=-1, keepdims=True)
    v = jnp.mean((x - m) ** 2, axis=-1, keepdims=True)
    return (x - m) * jax.lax.rsqrt(v + 1e-5) * g


def _embed_body(idx_ref, *refs):
    wrows = refs[:EG]
    wpe_ref = refs[EG]
    out_ref = refs[EG + 1]
    for k in range(EG):
        out_ref[k:k + 1, :] = wrows[k][0] + wpe_ref[k:k + 1, :]


def _embed(idx, wte, wpe):
    grid = (T // EG,)
    in_specs = [
        pl.BlockSpec((1, 1, C), functools.partial(
            lambda k, i, idx_ref: (idx_ref[0, EG * i + k], 0, 0), k))
        for k in range(EG)
    ] + [pl.BlockSpec((EG, C), lambda i, idx_ref: (i, 0))]
    return pl.pallas_call(
        _embed_body,
        grid_spec=pltpu.PrefetchScalarGridSpec(
            num_scalar_prefetch=1,
            grid=grid,
            in_specs=in_specs,
            out_specs=pl.BlockSpec((EG, C), lambda i, idx_ref: (i, 0)),
        ),
        out_shape=jax.ShapeDtypeStruct((T, C), jnp.float32),
    )(idx, *([wte.reshape(V, 1, C)] * EG), wpe)


def _ln_body(x_ref, g_ref, out_ref):
    out_ref[...] = _ln(x_ref[...], g_ref[...])


def _ln_call(x, g):
    return pl.pallas_call(
        _ln_body,
        out_shape=jax.ShapeDtypeStruct((T, C), jnp.float32),
    )(x, g.reshape(1, C))


def _qkv_body(x_ref, w_ref, out_ref):
    y = jax.lax.dot_general(
        x_ref[...].astype(jnp.bfloat16), w_ref[...], (((1,), (1,)), ((), ())),
        preferred_element_type=jnp.float32)
    y = y * jnp.where(pl.program_id(0) < NH // 2, 1.0 / (HD ** 0.5), 1.0)
    y = y.astype(jnp.bfloat16)
    out_ref[0] = y[:, :HD]
    out_ref[1] = y[:, HD:]


def _qkv(xn, w_attn_bf):
    grid = (3 * NH // 2,)
    return pl.pallas_call(
        _qkv_body,
        grid=grid,
        in_specs=[
            pl.BlockSpec((T, C), lambda j: (0, 0)),
            pl.BlockSpec((2 * HD, C), lambda j: (j, 0)),
        ],
        out_specs=pl.BlockSpec((2, T, HD), lambda j: (j, 0, 0)),
        out_shape=jax.ShapeDtypeStruct((3 * NH, T, HD), jnp.bfloat16),
    )(xn, w_attn_bf)


def _attn_body(q_ref, k_ref, v_ref, out_ref):
    r = pl.program_id(1)
    s = jax.lax.dot_general(q_ref[0], k_ref[0], (((1,), (1,)), ((), ())),
                            preferred_element_type=jnp.float32)
    rows = jax.lax.broadcasted_iota(jnp.int32, (RB, T), 0) + r * RB
    cols = jax.lax.broadcasted_iota(jnp.int32, (RB, T), 1)
    p = jnp.exp(jnp.minimum(s, 60.0))
    p = jnp.where(rows >= cols, p, 0.0)
    p = p / jnp.sum(p, axis=-1, keepdims=True)
    out_ref[0] = jnp.dot(p.astype(jnp.bfloat16), v_ref[0],
                         preferred_element_type=jnp.float32).astype(jnp.bfloat16)


def _attn(qkv):
    grid = (NH, T // RB)
    return pl.pallas_call(
        _attn_body,
        grid=grid,
        in_specs=[
            pl.BlockSpec((1, RB, HD), lambda h, r: (h, r, 0)),
            pl.BlockSpec((1, T, HD), lambda h, r: (NH + h, 0, 0)),
            pl.BlockSpec((1, T, HD), lambda h, r: (2 * NH + h, 0, 0)),
        ],
        out_specs=pl.BlockSpec((1, RB, HD), lambda h, r: (h, r, 0)),
        out_shape=jax.ShapeDtypeStruct((NH, T, HD), jnp.bfloat16),
    )(qkv, qkv, qkv)


def _proj_body(a_ref, w_ref, x_ref, g_ref, wg_ref, out_ref, xn_ref, lg_ref):
    h = pl.program_id(0)

    @pl.when(h == 0)
    def _():
        out_ref[...] = x_ref[...]

    out_ref[...] += jnp.dot(a_ref[0], w_ref[0],
                            preferred_element_type=jnp.float32)

    @pl.when(h == NH - 1)
    def _():
        xn = _ln(out_ref[...], g_ref[...])
        xn_ref[...] = xn
        lg_ref[...] = jax.lax.dot_general(
            xn, wg_ref[...], (((1,), (1,)), ((), ())),
            preferred_element_type=jnp.float32)


def _proj_router(a, wp_resh, x, g2, wg_pad):
    grid = (NH,)
    return pl.pallas_call(
        _proj_body,
        grid=grid,
        in_specs=[
            pl.BlockSpec((1, T, HD), lambda h: (h, 0, 0)),
            pl.BlockSpec((1, HD, C), lambda h: (h, 0, 0)),
            pl.BlockSpec((T, C), lambda h: (0, 0)),
            pl.BlockSpec((1, C), lambda h: (0, 0)),
            pl.BlockSpec((128, C), lambda h: (0, 0)),
        ],
        out_specs=(pl.BlockSpec((T, C), lambda h: (0, 0)),
                   pl.BlockSpec((T, C), lambda h: (0, 0)),
                   pl.BlockSpec((T, 128), lambda h: (0, 0))),
        out_shape=(jax.ShapeDtypeStruct((T, C), jnp.float32),
                   jax.ShapeDtypeStruct((T, C), jnp.float32),
                   jax.ShapeDtypeStruct((T, 128), jnp.float32)),
    )(a, wp_resh, x, g2.reshape(1, C), wg_pad)


def _ln2_body(x_ref, g_ref, wg_ref, xn_ref, lg_ref):
    xn = _ln(x_ref[...], g_ref[...])
    xn_ref[...] = xn
    lg_ref[...] = jax.lax.dot_general(
        xn, wg_ref[...], (((1,), (1,)), ((), ())),
        preferred_element_type=jnp.float32)


def _ln2_router(x, g, wg_pad):
    return pl.pallas_call(
        _ln2_body,
        out_shape=(jax.ShapeDtypeStruct((T, C), jnp.float32),
                   jax.ShapeDtypeStruct((T, 128), jnp.float32)),
    )(x, g.reshape(1, C), wg_pad)


def _routing_meta(logits):
    top_v, top_i = jax.lax.top_k(logits, TOPK)
    scores = jax.nn.softmax(top_v, axis=-1)
    e0, e1 = top_i[:, 0], top_i[:, 1]
    ear = jnp.arange(E, dtype=jnp.int32)
    oh0 = (e0[:, None] == ear).astype(jnp.int32)
    oh1 = (e1[:, None] == ear).astype(jnp.int32)
    inter = oh0 + oh1
    cumincl = jnp.cumsum(inter, axis=0)
    cumexcl = cumincl - inter
    counts = cumincl[-1]
    rank0 = jnp.take_along_axis(cumexcl, e0[:, None], 1)[:, 0]
    rank1 = jnp.take_along_axis(cumexcl, e1[:, None], 1)[:, 0]
    nb = (counts + BLKM - 1) // BLKM
    ends = jnp.cumsum(nb)
    blk_start_e = ends - nb
    nblk_total = ends[-1].astype(jnp.int32)
    pos0 = BLKM * blk_start_e[e0] + rank0
    pos1 = BLKM * blk_start_e[e1] + rank1
    tok = jnp.arange(T, dtype=jnp.int32)
    pos = jnp.concatenate([pos0, pos1])
    row_token = jnp.zeros((PADN,), jnp.int32).at[pos].set(
        jnp.concatenate([tok, tok]))
    bar = jnp.arange(NBLK, dtype=jnp.int32)
    e_of_blk = jnp.minimum(
        jnp.searchsorted(ends, bar, side='right'), E - 1).astype(jnp.int32)
    poscat = jnp.stack([pos0, pos1]).astype(jnp.int32)
    s0b = jnp.broadcast_to(scores[:, :1], (T, 128))
    s1b = jnp.broadcast_to(scores[:, 1:], (T, 128))
    return row_token, e_of_blk, nblk_total, poscat, s0b, s1b


SC_NW = 32


def _sc_gather(table, idx):
    n = idx.shape[0]
    d = table.shape[1]
    per = n // SC_NW
    nch = (per + 63) // 64
    chunks = [64] * (per // 64) + ([per % 64] if per % 64 else [])
    mesh = plsc.VectorSubcoreMesh(core_axis_name="c", subcore_axis_name="s")

    @functools.partial(
        pl.kernel, mesh=mesh,
        out_type=jax.ShapeDtypeStruct((n, d), jnp.float32),
        scratch_types=[
            pltpu.VMEM((nch, 64), jnp.int32),
            pltpu.VMEM((per, d), jnp.float32),
            pltpu.SemaphoreType.DMA,
        ],
    )
    def k(table_hbm, idx_hbm, out_hbm, idx_v, rows_v, sem):
        wid = lax.axis_index("s") * 2 + lax.axis_index("c")
        base = wid * per
        off = 0
        for ci, cl in enumerate(chunks):
            pltpu.sync_copy(idx_hbm.at[pl.ds(base + off, cl)],
                            idx_v.at[ci, pl.ds(0, cl)])
            off += cl
        copies = []
        off = 0
        for ci, cl in enumerate(chunks):
            copies.append(pltpu.async_copy(
                table_hbm.at[idx_v.at[ci, pl.ds(0, cl)]],
                rows_v.at[pl.ds(off, cl)], sem))
            off += cl
        for cp in copies:
            cp.wait()
        pltpu.sync_copy(rows_v, out_hbm.at[pl.ds(base, per)])

    return k(table, idx)


def _gffn_body(eblk_ref, nblk_ref, xg_ref, w1_ref, w2_ref, out_ref):
    b = pl.program_id(0)

    @pl.when(b < nblk_ref[0])
    def _():
        h = jax.lax.dot_general(
            xg_ref[...].astype(jnp.bfloat16), w1_ref[0],
            (((1,), (1,)), ((), ())),
            preferred_element_type=jnp.float32)
        h = 0.5 * h * (1.0 + jax.lax.erf(h * (2.0 ** -0.5)))
        y = jax.lax.dot_general(
            h.astype(jnp.bfloat16), w2_ref[0], (((1,), (1,)), ((), ())),
            preferred_element_type=jnp.float32)
        out_ref[...] = y


def _gffn(xg, w1, w2, e_of_blk, nblk_total):
    return pl.pallas_call(
        _gffn_body,
        grid_spec=pltpu.PrefetchScalarGridSpec(
            num_scalar_prefetch=2,
            grid=(NBLK,),
            in_specs=[
                pl.BlockSpec((BLKM, C), lambda b, eb, nb: (b, 0)),
                pl.BlockSpec((1, DFF, C), lambda b, eb, nb: (eb[b], 0, 0)),
                pl.BlockSpec((1, C, DFF), lambda b, eb, nb: (eb[b], 0, 0)),
            ],
            out_specs=pl.BlockSpec((BLKM, C), lambda b, eb, nb: (b, 0)),
        ),
        out_shape=jax.ShapeDtypeStruct((PADN, C), jnp.float32),
    )(e_of_blk, nblk_total.reshape(1), xg, w1, w2)


def _comb_body(x_ref, a_ref, b_ref, sa_ref, sb_ref, out_ref):
    out_ref[...] = (x_ref[...] + sa_ref[...][:, :1] * a_ref[...]
                    + sb_ref[...][:, :1] * b_ref[...])


def _comb(x, y01, s0b, s1b):
    return pl.pallas_call(
        _comb_body,
        grid=(1,),
        in_specs=[
            pl.BlockSpec((T, C), lambda i: (0, 0)),
            pl.BlockSpec((T, C), lambda i: (0, 0)),
            pl.BlockSpec((T, C), lambda i: (1, 0)),
            pl.BlockSpec((T, 128), lambda i: (0, 0)),
            pl.BlockSpec((T, 128), lambda i: (0, 0)),
        ],
        out_specs=pl.BlockSpec((T, C), lambda i: (0, 0)),
        out_shape=jax.ShapeDtypeStruct((T, C), jnp.float32),
    )(x, y01, y01, s0b, s1b)


def _add2_ln_body(a_ref, b_ref, g_ref, x_ref, xn_ref):
    x = a_ref[...] + b_ref[...]
    x_ref[...] = x
    xn_ref[...] = _ln(x, g_ref[...])


def _add2_ln(a, b, g):
    return pl.pallas_call(
        _add2_ln_body,
        out_shape=(jax.ShapeDtypeStruct((T, C), jnp.float32),
                   jax.ShapeDtypeStruct((T, C), jnp.float32)),
    )(a, b, g.reshape(1, C))


def _comb_ln_body(x_ref, a_ref, b_ref, sa_ref, sb_ref, g_ref, x2_ref, xn_ref):
    x2 = (x_ref[...] + sa_ref[...][:, :1] * a_ref[...]
          + sb_ref[...][:, :1] * b_ref[...])
    x2_ref[...] = x2
    xn_ref[...] = _ln(x2, g_ref[...])


def _comb_ln(x, y01, s0b, s1b, g):
    return pl.pallas_call(
        _comb_ln_body,
        grid=(1,),
        in_specs=[
            pl.BlockSpec((T, C), lambda i: (0, 0)),
            pl.BlockSpec((T, C), lambda i: (0, 0)),
            pl.BlockSpec((T, C), lambda i: (1, 0)),
            pl.BlockSpec((T, 128), lambda i: (0, 0)),
            pl.BlockSpec((T, 128), lambda i: (0, 0)),
            pl.BlockSpec((1, C), lambda i: (0, 0)),
        ],
        out_specs=(pl.BlockSpec((T, C), lambda i: (0, 0)),
                   pl.BlockSpec((T, C), lambda i: (0, 0))),
        out_shape=(jax.ShapeDtypeStruct((T, C), jnp.float32),
                   jax.ShapeDtypeStruct((T, C), jnp.float32)),
    )(x, y01, y01, s0b, s1b, g.reshape(1, C))


def _lm_body(x_ref, g_ref, wte_ref, out_ref):
    xn = _ln(x_ref[0], g_ref[...])
    out_ref[...] = jax.lax.dot_general(
        xn, wte_ref[...], (((1,), (1,)), ((), ())),
        preferred_element_type=jnp.float32)


def _lm_head(x, g, wte):
    grid = (V // VB,)
    return pl.pallas_call(
        _lm_body,
        grid=grid,
        in_specs=[
            pl.BlockSpec((1, 1, C), lambda i: (T - 1, 0, 0)),
            pl.BlockSpec((1, C), lambda i: (0, 0)),
            pl.BlockSpec((VB, C), lambda i: (i, 0)),
        ],
        out_specs=pl.BlockSpec((1, VB), lambda i: (0, i)),
        out_shape=jax.ShapeDtypeStruct((1, V), jnp.float32),
    )(x.reshape(T, 1, C), g.reshape(1, C), wte)


def kernel(idx, params):
    idx = idx.astype(jnp.int32)
    layers = params['layers']
    x, xn1 = _add2_ln(_sc_gather(params['wte'], idx.reshape(T)),
                      params['wpe'], layers[0]['ln1_g'])
    for li, lp in enumerate(layers):
        a = _attn(_qkv(xn1, lp['w_attn'].astype(jnp.bfloat16)))
        wp_resh = (lp['w_proj'].reshape(C, NH, HD).transpose(1, 2, 0)
                   .astype(jnp.bfloat16))
        wg_pad = jnp.zeros((128, C), jnp.float32).at[:E].set(lp['w_gate'])
        x, xn, logits_pad = _proj_router(a, wp_resh, x, lp['ln2_g'], wg_pad)
        logits = logits_pad[:, :E]
        row_token, e_of_blk, nblk_total, poscat, s0b, s1b = _routing_meta(logits)
        xg = _sc_gather(xn, row_token)
        yg = _gffn(xg, lp['w1'].astype(jnp.bfloat16),
                   lp['w2'].astype(jnp.bfloat16), e_of_blk, nblk_total)
        y01 = _sc_gather(yg, poscat.reshape(2 * T))
        if li + 1 < len(layers):
            x, xn1 = _comb_ln(x, y01, s0b, s1b, layers[li + 1]['ln1_g'])
        else:
            x = _comb(x, y01, s0b, s1b)
    logits = _lm_head(x, params['ln_f_g'], params['wte'])
    return logits.reshape(1, 1, V)


# speedup vs baseline: 1.0939x; 1.0939x over previous
import functools

import jax
import jax.numpy as jnp
from jax import lax
from jax.experimental import pallas as pl
from jax.experimental.pallas import tpu as pltpu
from jax.experimental.pallas import tpu_sc as plsc

V = 50304
T = 2048
C = 768
NH = 12
HD = C // NH
DFF = 4 * C
E = 8
TOPK = 2

EG = 8
RB = 1024
VB = 384

BLKM = 128
NBLK = (TOPK * T) // BLKM + E
PADN = NBLK * BLKM
MG = 16
CG = 8


def _ln(x, g):
    m = jnp.mean(x, axis=-1, keepdims=True)
    v = jnp.mean((x - m) ** 2, axis=-1, keepdims=True)
    return (x - m) * jax.lax.rsqrt(v + 1e-5) * g


def _embed_body(idx_ref, *refs):
    wrows = refs[:EG]
    wpe_ref = refs[EG]
    out_ref = refs[EG + 1]
    for k in range(EG):
        out_ref[k:k + 1, :] = wrows[k][0] + wpe_ref[k:k + 1, :]


def _embed(idx, wte, wpe):
    grid = (T // EG,)
    in_specs = [
        pl.BlockSpec((1, 1, C), functools.partial(
            lambda k, i, idx_ref: (idx_ref[0, EG * i + k], 0, 0), k))
        for k in range(EG)
    ] + [pl.BlockSpec((EG, C), lambda i, idx_ref: (i, 0))]
    return pl.pallas_call(
        _embed_body,
        grid_spec=pltpu.PrefetchScalarGridSpec(
            num_scalar_prefetch=1,
            grid=grid,
            in_specs=in_specs,
            out_specs=pl.BlockSpec((EG, C), lambda i, idx_ref: (i, 0)),
        ),
        out_shape=jax.ShapeDtypeStruct((T, C), jnp.float32),
    )(idx, *([wte.reshape(V, 1, C)] * EG), wpe)


def _ln_body(x_ref, g_ref, out_ref):
    out_ref[...] = _ln(x_ref[...], g_ref[...])


def _ln_call(x, g):
    return pl.pallas_call(
        _ln_body,
        out_shape=jax.ShapeDtypeStruct((T, C), jnp.float32),
    )(x, g.reshape(1, C))


def _qkv_body(x_ref, w_ref, out_ref):
    y = jax.lax.dot_general(
        x_ref[...].astype(jnp.bfloat16), w_ref[...], (((1,), (1,)), ((), ())),
        preferred_element_type=jnp.float32)
    y = y * jnp.where(pl.program_id(0) < NH // 2, 1.0 / (HD ** 0.5), 1.0)
    y = y.astype(jnp.bfloat16)
    out_ref[0] = y[:, :HD]
    out_ref[1] = y[:, HD:]


def _qkv(xn, w_attn_bf):
    grid = (3 * NH // 2,)
    return pl.pallas_call(
        _qkv_body,
        grid=grid,
        in_specs=[
            pl.BlockSpec((T, C), lambda j: (0, 0)),
            pl.BlockSpec((2 * HD, C), lambda j: (j, 0)),
        ],
        out_specs=pl.BlockSpec((2, T, HD), lambda j: (j, 0, 0)),
        out_shape=jax.ShapeDtypeStruct((3 * NH, T, HD), jnp.bfloat16),
    )(xn, w_attn_bf)


def _attn_body(q_ref, k_ref, v_ref, out_ref):
    r = pl.program_id(1)
    s = jax.lax.dot_general(q_ref[0], k_ref[0], (((1,), (1,)), ((), ())),
                            preferred_element_type=jnp.float32)
    rows = jax.lax.broadcasted_iota(jnp.int32, (RB, T), 0) + r * RB
    cols = jax.lax.broadcasted_iota(jnp.int32, (RB, T), 1)
    p = jnp.exp(jnp.minimum(s, 60.0))
    p = jnp.where(rows >= cols, p, 0.0)
    p = p / jnp.sum(p, axis=-1, keepdims=True)
    out_ref[0] = jnp.dot(p.astype(jnp.bfloat16), v_ref[0],
                         preferred_element_type=jnp.float32).astype(jnp.bfloat16)


def _attn(qkv):
    grid = (NH, T // RB)
    return pl.pallas_call(
        _attn_body,
        grid=grid,
        in_specs=[
            pl.BlockSpec((1, RB, HD), lambda h, r: (h, r, 0)),
            pl.BlockSpec((1, T, HD), lambda h, r: (NH + h, 0, 0)),
            pl.BlockSpec((1, T, HD), lambda h, r: (2 * NH + h, 0, 0)),
        ],
        out_specs=pl.BlockSpec((1, RB, HD), lambda h, r: (h, r, 0)),
        out_shape=jax.ShapeDtypeStruct((NH, T, HD), jnp.bfloat16),
    )(qkv, qkv, qkv)


def _proj_body(a_ref, w_ref, x_ref, g_ref, wg_ref, out_ref, xn_ref, lg_ref):
    h = pl.program_id(0)

    @pl.when(h == 0)
    def _():
        out_ref[...] = x_ref[...]

    out_ref[...] += jnp.dot(a_ref[0], w_ref[0],
                            preferred_element_type=jnp.float32)

    @pl.when(h == NH - 1)
    def _():
        xn = _ln(out_ref[...], g_ref[...])
        xn_ref[...] = xn
        lg_ref[...] = jax.lax.dot_general(
            xn, wg_ref[...], (((1,), (1,)), ((), ())),
            preferred_element_type=jnp.float32)


def _proj_router(a, wp_resh, x, g2, wg_pad):
    grid = (NH,)
    return pl.pallas_call(
        _proj_body,
        grid=grid,
        in_specs=[
            pl.BlockSpec((1, T, HD), lambda h: (h, 0, 0)),
            pl.BlockSpec((1, HD, C), lambda h: (h, 0, 0)),
            pl.BlockSpec((T, C), lambda h: (0, 0)),
            pl.BlockSpec((1, C), lambda h: (0, 0)),
            pl.BlockSpec((128, C), lambda h: (0, 0)),
        ],
        out_specs=(pl.BlockSpec((T, C), lambda h: (0, 0)),
                   pl.BlockSpec((T, C), lambda h: (0, 0)),
                   pl.BlockSpec((T, 128), lambda h: (0, 0))),
        out_shape=(jax.ShapeDtypeStruct((T, C), jnp.float32),
                   jax.ShapeDtypeStruct((T, C), jnp.float32),
                   jax.ShapeDtypeStruct((T, 128), jnp.float32)),
    )(a, wp_resh, x, g2.reshape(1, C), wg_pad)


def _ln2_body(x_ref, g_ref, wg_ref, xn_ref, lg_ref):
    xn = _ln(x_ref[...], g_ref[...])
    xn_ref[...] = xn
    lg_ref[...] = jax.lax.dot_general(
        xn, wg_ref[...], (((1,), (1,)), ((), ())),
        preferred_element_type=jnp.float32)


def _ln2_router(x, g, wg_pad):
    return pl.pallas_call(
        _ln2_body,
        out_shape=(jax.ShapeDtypeStruct((T, C), jnp.float32),
                   jax.ShapeDtypeStruct((T, 128), jnp.float32)),
    )(x, g.reshape(1, C), wg_pad)


def _routing_meta(logits):
    top_v, top_i = jax.lax.top_k(logits, TOPK)
    scores = jax.nn.softmax(top_v, axis=-1)
    e0, e1 = top_i[:, 0], top_i[:, 1]
    ear = jnp.arange(E, dtype=jnp.int32)
    oh0 = (e0[:, None] == ear).astype(jnp.int32)
    oh1 = (e1[:, None] == ear).astype(jnp.int32)
    inter = oh0 + oh1
    cumincl = jnp.cumsum(inter, axis=0)
    cumexcl = cumincl - inter
    counts = cumincl[-1]
    rank0 = jnp.take_along_axis(cumexcl, e0[:, None], 1)[:, 0]
    rank1 = jnp.take_along_axis(cumexcl, e1[:, None], 1)[:, 0]
    nb = (counts + BLKM - 1) // BLKM
    ends = jnp.cumsum(nb)
    blk_start_e = ends - nb
    nblk_total = ends[-1].astype(jnp.int32)
    pos0 = BLKM * blk_start_e[e0] + rank0
    pos1 = BLKM * blk_start_e[e1] + rank1
    tok = jnp.arange(T, dtype=jnp.int32)
    pos = jnp.concatenate([pos0, pos1])
    row_token = (jnp.arange(PADN, dtype=jnp.int32) % T).at[pos].set(
        jnp.concatenate([tok, tok]))
    bar = jnp.arange(NBLK, dtype=jnp.int32)
    e_of_blk = jnp.minimum(
        jnp.searchsorted(ends, bar, side='right'), E - 1).astype(jnp.int32)
    poscat = jnp.stack([pos0, pos1]).astype(jnp.int32)
    s0b = jnp.broadcast_to(scores[:, :1], (T, 128))
    s1b = jnp.broadcast_to(scores[:, 1:], (T, 128))
    return row_token, e_of_blk, nblk_total, poscat, s0b, s1b


SC_NW = 32


def _sc_gather(table, idx):
    n = idx.shape[0]
    d = table.shape[1]
    per = n // SC_NW
    nch = (per + 63) // 64
    chunks = [64] * (per // 64) + ([per % 64] if per % 64 else [])
    mesh = plsc.VectorSubcoreMesh(core_axis_name="c", subcore_axis_name="s")

    @functools.partial(
        pl.kernel, mesh=mesh,
        out_type=jax.ShapeDtypeStruct((n, d), jnp.float32),
        scratch_types=[
            pltpu.VMEM((nch, 64), jnp.int32),
            pltpu.VMEM((per, d), jnp.float32),
            pltpu.SemaphoreType.DMA,
        ],
    )
    def k(table_hbm, idx_hbm, out_hbm, idx_v, rows_v, sem):
        wid = lax.axis_index("s") * 2 + lax.axis_index("c")
        base = wid * per
        off = 0
        for ci, cl in enumerate(chunks):
            pltpu.sync_copy(idx_hbm.at[pl.ds(base + off, cl)],
                            idx_v.at[ci, pl.ds(0, cl)])
            off += cl
        copies = []
        off = 0
        for ci, cl in enumerate(chunks):
            copies.append(pltpu.async_copy(
                table_hbm.at[idx_v.at[ci, pl.ds(0, cl)]],
                rows_v.at[pl.ds(off, cl)], sem))
            off += cl
        for cp in copies:
            cp.wait()
        pltpu.sync_copy(rows_v, out_hbm.at[pl.ds(base, per)])

    return k(table, idx)


def _gffn_body(eblk_ref, nblk_ref, xg_ref, w1_ref, w2_ref, out_ref):
    b = pl.program_id(0)

    @pl.when(b < nblk_ref[0])
    def _():
        h = jax.lax.dot_general(
            xg_ref[...].astype(jnp.bfloat16), w1_ref[0],
            (((1,), (1,)), ((), ())),
            preferred_element_type=jnp.float32)
        h = 0.5 * h * (1.0 + jax.lax.erf(h * (2.0 ** -0.5)))
        y = jax.lax.dot_general(
            h.astype(jnp.bfloat16), w2_ref[0], (((1,), (1,)), ((), ())),
            preferred_element_type=jnp.float32)
        out_ref[...] = y


def _gffn(xg, w1, w2, e_of_blk, nblk_total):
    return pl.pallas_call(
        _gffn_body,
        grid_spec=pltpu.PrefetchScalarGridSpec(
            num_scalar_prefetch=2,
            grid=(NBLK,),
            in_specs=[
                pl.BlockSpec((BLKM, C), lambda b, eb, nb: (b, 0)),
                pl.BlockSpec((1, DFF, C), lambda b, eb, nb: (eb[b], 0, 0)),
                pl.BlockSpec((1, C, DFF), lambda b, eb, nb: (eb[b], 0, 0)),
            ],
            out_specs=pl.BlockSpec((BLKM, C), lambda b, eb, nb: (b, 0)),
        ),
        out_shape=jax.ShapeDtypeStruct((PADN, C), jnp.float32),
    )(e_of_blk, nblk_total.reshape(1), xg, w1, w2)


def _comb_body(x_ref, a_ref, b_ref, sa_ref, sb_ref, out_ref):
    out_ref[...] = (x_ref[...] + sa_ref[...][:, :1] * a_ref[...]
                    + sb_ref[...][:, :1] * b_ref[...])


def _comb(x, y01, s0b, s1b):
    return pl.pallas_call(
        _comb_body,
        grid=(1,),
        in_specs=[
            pl.BlockSpec((T, C), lambda i: (0, 0)),
            pl.BlockSpec((T, C), lambda i: (0, 0)),
            pl.BlockSpec((T, C), lambda i: (1, 0)),
            pl.BlockSpec((T, 128), lambda i: (0, 0)),
            pl.BlockSpec((T, 128), lambda i: (0, 0)),
        ],
        out_specs=pl.BlockSpec((T, C), lambda i: (0, 0)),
        out_shape=jax.ShapeDtypeStruct((T, C), jnp.float32),
    )(x, y01, y01, s0b, s1b)


def _add2_ln_body(a_ref, b_ref, g_ref, x_ref, xn_ref):
    x = a_ref[...] + b_ref[...]
    x_ref[...] = x
    xn_ref[...] = _ln(x, g_ref[...])


def _add2_ln(a, b, g):
    return pl.pallas_call(
        _add2_ln_body,
        out_shape=(jax.ShapeDtypeStruct((T, C), jnp.float32),
                   jax.ShapeDtypeStruct((T, C), jnp.float32)),
    )(a, b, g.reshape(1, C))


def _comb_ln_body(x_ref, a_ref, b_ref, sa_ref, sb_ref, g_ref, x2_ref, xn_ref):
    x2 = (x_ref[...] + sa_ref[...][:, :1] * a_ref[...]
          + sb_ref[...][:, :1] * b_ref[...])
    x2_ref[...] = x2
    xn_ref[...] = _ln(x2, g_ref[...])


def _comb_ln(x, y01, s0b, s1b, g):
    return pl.pallas_call(
        _comb_ln_body,
        grid=(1,),
        in_specs=[
            pl.BlockSpec((T, C), lambda i: (0, 0)),
            pl.BlockSpec((T, C), lambda i: (0, 0)),
            pl.BlockSpec((T, C), lambda i: (1, 0)),
            pl.BlockSpec((T, 128), lambda i: (0, 0)),
            pl.BlockSpec((T, 128), lambda i: (0, 0)),
            pl.BlockSpec((1, C), lambda i: (0, 0)),
        ],
        out_specs=(pl.BlockSpec((T, C), lambda i: (0, 0)),
                   pl.BlockSpec((T, C), lambda i: (0, 0))),
        out_shape=(jax.ShapeDtypeStruct((T, C), jnp.float32),
                   jax.ShapeDtypeStruct((T, C), jnp.float32)),
    )(x, y01, y01, s0b, s1b, g.reshape(1, C))


def _lm_body(x_ref, g_ref, wte_ref, out_ref):
    xn = _ln(x_ref[0], g_ref[...])
    out_ref[...] = jax.lax.dot_general(
        xn, wte_ref[...], (((1,), (1,)), ((), ())),
        preferred_element_type=jnp.float32)


def _lm_head(x, g, wte):
    grid = (V // VB,)
    return pl.pallas_call(
        _lm_body,
        grid=grid,
        in_specs=[
            pl.BlockSpec((1, 1, C), lambda i: (T - 1, 0, 0)),
            pl.BlockSpec((1, C), lambda i: (0, 0)),
            pl.BlockSpec((VB, C), lambda i: (i, 0)),
        ],
        out_specs=pl.BlockSpec((1, VB), lambda i: (0, i)),
        out_shape=jax.ShapeDtypeStruct((1, V), jnp.float32),
    )(x.reshape(T, 1, C), g.reshape(1, C), wte)


def kernel(idx, params):
    idx = idx.astype(jnp.int32)
    layers = params['layers']
    x, xn1 = _add2_ln(_sc_gather(params['wte'], idx.reshape(T)),
                      params['wpe'], layers[0]['ln1_g'])
    for li, lp in enumerate(layers):
        a = _attn(_qkv(xn1, lp['w_attn'].astype(jnp.bfloat16)))
        wp_resh = (lp['w_proj'].reshape(C, NH, HD).transpose(1, 2, 0)
                   .astype(jnp.bfloat16))
        wg_pad = jnp.zeros((128, C), jnp.float32).at[:E].set(lp['w_gate'])
        x, xn, logits_pad = _proj_router(a, wp_resh, x, lp['ln2_g'], wg_pad)
        logits = logits_pad[:, :E]
        row_token, e_of_blk, nblk_total, poscat, s0b, s1b = _routing_meta(logits)
        xg = _sc_gather(xn, row_token)
        yg = _gffn(xg, lp['w1'].astype(jnp.bfloat16),
                   lp['w2'].astype(jnp.bfloat16), e_of_blk, nblk_total)
        y01 = _sc_gather(yg, poscat.reshape(2 * T))
        if li + 1 < len(layers):
            x, xn1 = _comb_ln(x, y01, s0b, s1b, layers[li + 1]['ln1_g'])
        else:
            x = _comb(x, y01, s0b, s1b)
    logits = _lm_head(x, params['ln_f_g'], params['wte'])
    return logits.reshape(1, 1, V)
